# baseline (speedup 1.0000x reference)
import functools

import jax
import jax.numpy as jnp
from jax import lax
from jax.experimental import pallas as pl

B = 16
N = 256
D = 128
MD = 64
H = 128
TOTAL_METRICS = 4096
GROUP = 64

TILE_T = 512
INST_PER_TILE = (TILE_T // GROUP) * 4


def _fused_kernel(h_ref, emb_ref, w1h_ref, w1e_ref, b1_ref, w2_ref, b2_ref,
                  w3_ref, b3_ref, out_ref):
    h = h_ref[0]
    emb = emb_ref[0]

    hw = jnp.dot(h, w1h_ref[...], preferred_element_type=jnp.float32,
                 precision=lax.Precision.HIGHEST)
    ew = jnp.dot(emb, w1e_ref[...], preferred_element_type=jnp.float32,
                 precision=lax.Precision.HIGHEST)

    row = lax.broadcasted_iota(jnp.int32, (TILE_T, INST_PER_TILE), 0)
    col = lax.broadcasted_iota(jnp.int32, (TILE_T, INST_PER_TILE), 1)
    off = row % GROUP
    sub = ((off >= 8).astype(jnp.int32) + (off >= 24).astype(jnp.int32)
           + (off >= 48).astype(jnp.int32))
    inst = 4 * (row // GROUP) + sub
    onehot = (inst == col).astype(jnp.float32)

    x1 = jnp.dot(onehot, hw, preferred_element_type=jnp.float32,
                 precision=lax.Precision.HIGHEST) + ew + b1_ref[0]
    x1 = jnp.maximum(x1, 0.0)
    x2 = jnp.dot(x1, w2_ref[...], preferred_element_type=jnp.float32,
                 precision=lax.Precision.HIGHEST) + b2_ref[0]
    x2 = jnp.maximum(x2, 0.0)
    res = jnp.sum(x2 * w3_ref[0], axis=1) + b3_ref[0, 0]
    out_ref[...] = res.reshape(1, 1, 1, TILE_T)


@functools.partial(jax.jit)
def kernel(h_instances, metric_embeddings, W1, b1, W2, b2, W3, b3):
    w1h = W1[:D]
    w1e = W1[D:]
    b1r = b1.reshape(1, H)
    b2r = b2.reshape(1, H // 2)
    w3r = W3.reshape(1, H // 2)
    b3r = b3.reshape(1, 1)

    n_tiles = TOTAL_METRICS // TILE_T
    grid = (B, n_tiles)

    return pl.pallas_call(
        _fused_kernel,
        grid=grid,
        in_specs=[
            pl.BlockSpec((1, INST_PER_TILE, D), lambda b, t: (b, t, 0)),
            pl.BlockSpec((1, TILE_T, MD), lambda b, t: (b, t, 0)),
            pl.BlockSpec((D, H), lambda b, t: (0, 0)),
            pl.BlockSpec((MD, H), lambda b, t: (0, 0)),
            pl.BlockSpec((1, H), lambda b, t: (0, 0)),
            pl.BlockSpec((H, H // 2), lambda b, t: (0, 0)),
            pl.BlockSpec((1, H // 2), lambda b, t: (0, 0)),
            pl.BlockSpec((1, H // 2), lambda b, t: (0, 0)),
            pl.BlockSpec((1, 1), lambda b, t: (0, 0)),
        ],
        out_specs=pl.BlockSpec((1, 1, 1, TILE_T), lambda b, t: (b, t, 0, 0)),
        out_shape=jax.ShapeDtypeStruct((B, n_tiles, 1, TILE_T), jnp.float32),
    )(h_instances, metric_embeddings, w1h, w1e, b1r, W2, b2r, w3r,
      b3r).reshape(B, TOTAL_METRICS)
